# baseline (speedup 1.0000x reference)
import functools

import jax
import jax.numpy as jnp
from jax import lax
from jax.experimental import pallas as pl
from jax.experimental.pallas import tpu as pltpu
from jax.experimental.pallas import tpu_sc as plsc

D = 16
NLAB = 5
DELTA_V = 0.5
DELTA_D = 3.0

BS = 8
P = 384 * 384
NC = 2
NS = 16
B_PER_CORE = BS // NC
P_PER_TILE = P // NS
CHUNK = 2304
NCHUNK = P_PER_TILE // CHUNK
NGROUP = CHUNK // 16

SUM_STRIDE = 80
ROW = 128


def _i16(x):
    return jnp.zeros((16,), jnp.int32) + x


def _f16(x):
    return jnp.zeros((16,), jnp.float32) + x


def _sqrt16(x):
    i = plsc.bitcast(x, jnp.int32)
    i = _i16(0x5F3759DF) - (i >> 1)
    y = plsc.bitcast(i, jnp.float32)
    for _ in range(3):
        y = y * (1.5 - 0.5 * x * y * y)
    return x * y


def _body(emb_hbm, seg_hbm, out_hbm,
          emb_buf, seg_buf, sums_tab, cnt_tab, var_tab,
          pub, pub2, shared1, shared2, coll, coll2, tot,
          means_tabs, pres_tab, outbuf):
    cid = lax.axis_index("c")
    sid = lax.axis_index("s")
    io = lax.iota(jnp.int32, 16)
    iof = io.astype(jnp.float32)
    z16 = jnp.zeros((16,), jnp.float32)
    one16 = jnp.ones((16,), jnp.float32)
    pix_base = sid * P_PER_TILE

    def stream_chunk(b, k):
        bb = cid * B_PER_CORE + b
        off = pix_base + k * CHUNK
        pltpu.sync_copy(emb_hbm.at[bb, :, pl.ds(off, CHUNK)], emb_buf)
        pltpu.sync_copy(seg_hbm.at[bb, pl.ds(off, CHUNK)], seg_buf)

    def batch1(b, carry):
        for i in range(SUM_STRIDE * 16 // 16):
            sums_tab[pl.ds(i * 16, 16)] = z16
        for i in range(16):
            cnt_tab[pl.ds(i * 16, 16)] = z16

        def chunk1(k, c):
            stream_chunk(b, k)

            def grp(g, c2):
                s = seg_buf[pl.ds(g * 16, 16)]
                base = io * SUM_STRIDE + s * 16
                for d in range(D):
                    e = emb_buf[d, pl.ds(g * 16, 16)]
                    plsc.addupdate_scatter(sums_tab, [base + d], e)
                plsc.addupdate_scatter(cnt_tab, [io * 16 + s], one16)
                return c2

            return lax.fori_loop(0, NGROUP, grp, c)

        lax.fori_loop(0, NCHUNK, chunk1, 0)

        for l in range(NLAB):
            acc = z16
            for lane in range(16):
                acc = acc + sums_tab[pl.ds(lane * SUM_STRIDE + l * 16, 16)]
            pub[pl.ds(b * ROW + l * 16, 16)] = acc
        acc = z16
        for lane in range(16):
            acc = acc + cnt_tab[pl.ds(lane * 16, 16)]
        pub[pl.ds(b * ROW + 80, 16)] = acc
        return carry

    lax.fori_loop(0, B_PER_CORE, batch1, 0)
    pltpu.sync_copy(pub, shared1.at[sid])
    plsc.subcore_barrier()
    pltpu.sync_copy(shared1, coll)

    def batchm(b, carry):
        for w in range(6):
            acc = z16
            for t in range(16):
                acc = acc + coll[t, pl.ds(b * ROW + w * 16, 16)]
            tot[pl.ds(b * ROW + w * 16, 16)] = acc
        means_tabs[pl.ds(b * ROW, 16)] = z16
        for l in range(1, NLAB):
            cnt = plsc.load_gather(tot, [_i16(b * ROW + 80 + l)])
            safe = jnp.maximum(cnt, 1.0)
            mean_l = tot[pl.ds(b * ROW + l * 16, 16)] / safe
            means_tabs[pl.ds(b * ROW + l * 16, 16)] = mean_l
        return carry

    lax.fori_loop(0, B_PER_CORE, batchm, 0)

    def batch2(b, carry):
        for i in range(16):
            var_tab[pl.ds(i * 16, 16)] = z16

        def chunk2(k, c):
            stream_chunk(b, k)

            def grp2(g, c2):
                s = seg_buf[pl.ds(g * 16, 16)]
                mbase = s * 16 + b * ROW
                acc = z16
                for d in range(D):
                    e = emb_buf[d, pl.ds(g * 16, 16)]
                    m = plsc.load_gather(means_tabs, [mbase + d])
                    t = e - m
                    acc = acc + t * t
                nsq = jnp.maximum(acc, 1e-12)
                n = _sqrt16(nsq)
                r = jnp.maximum(n - DELTA_V, 0.0)
                contrib = jnp.where(s > 0, r * r, 0.0)
                plsc.addupdate_scatter(var_tab, [io * 16 + s], contrib)
                return c2

            return lax.fori_loop(0, NGROUP, grp2, c)

        lax.fori_loop(0, NCHUNK, chunk2, 0)
        acc = z16
        for lane in range(16):
            acc = acc + var_tab[pl.ds(lane * 16, 16)]
        pub2[pl.ds(b * 16, 16)] = acc
        return carry

    lax.fori_loop(0, B_PER_CORE, batch2, 0)
    pltpu.sync_copy(pub2, shared2.at[sid])
    plsc.subcore_barrier()
    pltpu.sync_copy(shared2, coll2)

    def _splat_sum(vec, lanes):
        pres_tab[...] = vec
        acc = plsc.load_gather(pres_tab, [_i16(lanes[0])])
        for k in lanes[1:]:
            acc = acc + plsc.load_gather(pres_tab, [_i16(k)])
        return acc

    def fin(b, tot_loss):
        varvec = z16
        for t in range(16):
            varvec = varvec + coll2[t, pl.ds(b * 16, 16)]
        cvec = tot[pl.ds(b * ROW + 80, 16)]
        lane_ok = jnp.logical_and(io >= 1, io <= NLAB - 1)
        pres = jnp.where(jnp.logical_and(cvec > 0, lane_ok), 1.0, 0.0)
        vpl = varvec / jnp.maximum(cvec, 1.0)
        nl = _splat_sum(pres, list(range(1, NLAB)))
        bv = _splat_sum(vpl * pres, list(range(1, NLAB)))
        batch_var = jnp.where(nl > 0.0, bv / jnp.maximum(nl, 1.0), 0.0)

        prvec = pres
        pres_tab[...] = prvec
        iv = (io >> 2) + 1
        jv = (io & 3) + 1
        pi = plsc.load_gather(pres_tab, [iv])
        pj = plsc.load_gather(pres_tab, [jv])
        dacc = z16
        for d in range(D):
            mi = plsc.load_gather(means_tabs, [iv * 16 + (b * ROW + d)])
            mj = plsc.load_gather(means_tabs, [jv * 16 + (b * ROW + d)])
            t = mi - mj
            dacc = dacc + t * t
        dist = _sqrt16(jnp.maximum(dacc, 1e-12))
        pm = pi * pj * jnp.where(iv == jv, 0.0, 1.0)
        t2 = jnp.maximum(DELTA_D - dist, 0.0)
        ds_ = _splat_sum(t2 * t2 * pm, list(range(16)))
        batch_dist = jnp.where(
            nl > 1.0, ds_ / jnp.maximum(nl * (nl - 1.0), 1.0) / 2.0, 0.0)
        return tot_loss + batch_var + batch_dist

    total = lax.fori_loop(0, B_PER_CORE, fin, z16)
    total = total * (1.0 / BS)
    outbuf[...] = jnp.where(io == 0, total, 0.0)

    @pl.when(sid == 0)
    def _():
        pltpu.sync_copy(outbuf, out_hbm.at[cid])


@functools.partial(
    pl.kernel,
    out_type=jax.ShapeDtypeStruct((NC, 16), jnp.float32),
    mesh=plsc.VectorSubcoreMesh(core_axis_name="c", subcore_axis_name="s"),
    compiler_params=pltpu.CompilerParams(needs_layout_passes=False),
    scratch_types=[
        pltpu.VMEM((D, CHUNK), jnp.float32),
        pltpu.VMEM((CHUNK,), jnp.int32),
        pltpu.VMEM((16 * SUM_STRIDE,), jnp.float32),
        pltpu.VMEM((16 * 16,), jnp.float32),
        pltpu.VMEM((16 * 16,), jnp.float32),
        pltpu.VMEM((B_PER_CORE * ROW,), jnp.float32),
        pltpu.VMEM((B_PER_CORE * 16,), jnp.float32),
        pltpu.VMEM_SHARED((NS, B_PER_CORE * ROW), jnp.float32),
        pltpu.VMEM_SHARED((NS, B_PER_CORE * 16), jnp.float32),
        pltpu.VMEM((NS, B_PER_CORE * ROW), jnp.float32),
        pltpu.VMEM((NS, B_PER_CORE * 16), jnp.float32),
        pltpu.VMEM((B_PER_CORE * ROW,), jnp.float32),
        pltpu.VMEM((B_PER_CORE * ROW,), jnp.float32),
        pltpu.VMEM((16,), jnp.float32),
        pltpu.VMEM((16,), jnp.float32),
    ],
)
def _disc_loss_sc(emb_hbm, seg_hbm, out_hbm, *scratch):
    _body(emb_hbm, seg_hbm, out_hbm, *scratch)


def kernel(embedding, seg_gt):
    emb = embedding.reshape(BS, D, P)
    seg = seg_gt.reshape(BS, P).astype(jnp.int32)
    out = _disc_loss_sc(emb, seg)
    return out[0, 0] + out[1, 0]


# speedup vs baseline: 1.5129x; 1.5129x over previous
import functools

import jax
import jax.numpy as jnp
from jax import lax
from jax.experimental import pallas as pl
from jax.experimental.pallas import tpu as pltpu
from jax.experimental.pallas import tpu_sc as plsc

D = 16
NLAB = 5
DELTA_V = 0.5
DELTA_D = 3.0

BS = 8
P = 384 * 384
NC = 2
NS = 16
B_PER_CORE = BS // NC
P_PER_TILE = P // NS
CHUNK = 2304
NCHUNK = P_PER_TILE // CHUNK
NT = B_PER_CORE * NCHUNK
NGROUP = CHUNK // 16
UNROLL = 2

SUM_STRIDE = 80
SUMS_B = 16 * SUM_STRIDE
ROW = 128


def _i16(x):
    return jnp.zeros((16,), jnp.int32) + x


def _sqrt16(x):
    i = plsc.bitcast(x, jnp.int32)
    i = _i16(0x5F3759DF) - (i >> 1)
    y = plsc.bitcast(i, jnp.float32)
    for _ in range(3):
        y = y * (1.5 - 0.5 * x * y * y)
    return x * y


def _body(emb_hbm, seg_hbm, out_hbm,
          emb_a, emb_b, seg_a, seg_b, sums_tab, cnt_tab, var_tab,
          pub, pub2, shared1, shared2, coll, coll2, tot,
          means_tabs, pres_tab, outbuf, sem_a, sem_b):
    cid = lax.axis_index("c")
    sid = lax.axis_index("s")
    io = lax.iota(jnp.int32, 16)
    z16 = jnp.zeros((16,), jnp.float32)
    one16 = jnp.ones((16,), jnp.float32)
    pix_base = sid * P_PER_TILE

    def start_copy(t, eb, sb, sem):
        b = t >> 2
        k = t & (NCHUNK - 1)
        bb = cid * B_PER_CORE + b
        off = pix_base + k * CHUNK
        pltpu.async_copy(emb_hbm.at[bb, :, pl.ds(off, CHUNK)], eb, sem)
        pltpu.async_copy(seg_hbm.at[bb, pl.ds(off, CHUNK)], sb, sem)

    def wait_copy(eb, sb, sem):
        pltpu.make_async_copy(emb_hbm.at[0, :, pl.ds(0, CHUNK)], eb, sem).wait()
        pltpu.make_async_copy(seg_hbm.at[0, pl.ds(0, CHUNK)], sb, sem).wait()

    def ring(do_groups, drain):
        def pair(m, carry):
            t0 = 2 * m
            t1 = t0 + 1
            start_copy(t1, emb_b, seg_b, sem_b)
            wait_copy(emb_a, seg_a, sem_a)
            do_groups(t0, emb_a, seg_a)
            start_copy((t1 + 1) & (NT - 1), emb_a, seg_a, sem_a)
            wait_copy(emb_b, seg_b, sem_b)
            do_groups(t1, emb_b, seg_b)
            return carry
        lax.fori_loop(0, NT // 2, pair, 0)
        if drain:
            wait_copy(emb_a, seg_a, sem_a)

    start_copy(0, emb_a, seg_a, sem_a)

    def zinit(ref, nwords):
        def zi(i, c):
            ref[pl.ds(i * 16, 16)] = z16
            return c
        lax.fori_loop(0, nwords // 16, zi, 0)

    zinit(sums_tab, B_PER_CORE * SUMS_B)
    zinit(cnt_tab, B_PER_CORE * 256)
    zinit(var_tab, B_PER_CORE * 256)

    def do_groups1(t, eb, sb):
        b = t >> 2
        sbase = io * SUM_STRIDE + b * SUMS_B
        cbase = io * 16 + b * 256

        @plsc.parallel_loop(0, NGROUP, unroll=UNROLL)
        def _grp(g):
            s = sb[pl.ds(g * 16, 16)]
            base = sbase + s * 16
            for d in range(D):
                e = eb[d, pl.ds(g * 16, 16)]
                plsc.addupdate_scatter(sums_tab, [base + d], e)
            plsc.addupdate_scatter(cnt_tab, [cbase + s], one16)

    ring(do_groups1, drain=False)

    def red1(b, carry):
        for l in range(NLAB):
            acc = z16
            for lane in range(16):
                acc = acc + sums_tab[
                    pl.ds(b * SUMS_B + lane * SUM_STRIDE + l * 16, 16)]
            pub[pl.ds(b * ROW + l * 16, 16)] = acc
        acc = z16
        for lane in range(16):
            acc = acc + cnt_tab[pl.ds(b * 256 + lane * 16, 16)]
        pub[pl.ds(b * ROW + 80, 16)] = acc
        return carry

    lax.fori_loop(0, B_PER_CORE, red1, 0)
    pltpu.sync_copy(pub, shared1.at[sid])
    plsc.subcore_barrier()
    pltpu.sync_copy(shared1, coll)

    def batchm(b, carry):
        for w in range(6):
            acc = z16
            for t in range(16):
                acc = acc + coll[t, pl.ds(b * ROW + w * 16, 16)]
            tot[pl.ds(b * ROW + w * 16, 16)] = acc
        means_tabs[pl.ds(b * ROW, 16)] = z16
        for l in range(1, NLAB):
            cnt = plsc.load_gather(tot, [_i16(b * ROW + 80 + l)])
            safe = jnp.maximum(cnt, 1.0)
            mean_l = tot[pl.ds(b * ROW + l * 16, 16)] / safe
            means_tabs[pl.ds(b * ROW + l * 16, 16)] = mean_l
        return carry

    lax.fori_loop(0, B_PER_CORE, batchm, 0)

    def do_groups2(t, eb, sb):
        b = t >> 2
        vbase = io * 16 + b * 256
        mb = b * ROW

        @plsc.parallel_loop(0, NGROUP, unroll=UNROLL)
        def _grp(g):
            s = sb[pl.ds(g * 16, 16)]
            mbase = s * 16 + mb
            acc = z16
            for d in range(D):
                e = eb[d, pl.ds(g * 16, 16)]
                m = plsc.load_gather(means_tabs, [mbase + d])
                tt = e - m
                acc = acc + tt * tt
            nsq = jnp.maximum(acc, 1e-12)
            n = _sqrt16(nsq)
            r = jnp.maximum(n - DELTA_V, 0.0)
            contrib = jnp.where(s > 0, r * r, 0.0)
            plsc.addupdate_scatter(var_tab, [vbase + s], contrib)

    ring(do_groups2, drain=True)

    def red2(b, carry):
        acc = z16
        for lane in range(16):
            acc = acc + var_tab[pl.ds(b * 256 + lane * 16, 16)]
        pub2[pl.ds(b * 16, 16)] = acc
        return carry

    lax.fori_loop(0, B_PER_CORE, red2, 0)
    pltpu.sync_copy(pub2, shared2.at[sid])
    plsc.subcore_barrier()
    pltpu.sync_copy(shared2, coll2)

    def _splat_sum(vec, lanes):
        pres_tab[...] = vec
        acc = plsc.load_gather(pres_tab, [_i16(lanes[0])])
        for k in lanes[1:]:
            acc = acc + plsc.load_gather(pres_tab, [_i16(k)])
        return acc

    def fin(b, tot_loss):
        varvec = z16
        for t in range(16):
            varvec = varvec + coll2[t, pl.ds(b * 16, 16)]
        cvec = tot[pl.ds(b * ROW + 80, 16)]
        lane_ok = jnp.logical_and(io >= 1, io <= NLAB - 1)
        pres = jnp.where(jnp.logical_and(cvec > 0, lane_ok), 1.0, 0.0)
        vpl = varvec / jnp.maximum(cvec, 1.0)
        nl = _splat_sum(pres, list(range(1, NLAB)))
        bv = _splat_sum(vpl * pres, list(range(1, NLAB)))
        batch_var = jnp.where(nl > 0.0, bv / jnp.maximum(nl, 1.0), 0.0)

        prvec = pres
        pres_tab[...] = prvec
        iv = (io >> 2) + 1
        jv = (io & 3) + 1
        pi = plsc.load_gather(pres_tab, [iv])
        pj = plsc.load_gather(pres_tab, [jv])
        dacc = z16
        for d in range(D):
            mi = plsc.load_gather(means_tabs, [iv * 16 + (b * ROW + d)])
            mj = plsc.load_gather(means_tabs, [jv * 16 + (b * ROW + d)])
            t = mi - mj
            dacc = dacc + t * t
        dist = _sqrt16(jnp.maximum(dacc, 1e-12))
        pm = pi * pj * jnp.where(iv == jv, 0.0, 1.0)
        t2 = jnp.maximum(DELTA_D - dist, 0.0)
        ds_ = _splat_sum(t2 * t2 * pm, list(range(16)))
        batch_dist = jnp.where(
            nl > 1.0, ds_ / jnp.maximum(nl * (nl - 1.0), 1.0) / 2.0, 0.0)
        return tot_loss + batch_var + batch_dist

    total = lax.fori_loop(0, B_PER_CORE, fin, z16)
    total = total * (1.0 / BS)
    outbuf[...] = jnp.where(io == 0, total, 0.0)

    @pl.when(sid == 0)
    def _():
        pltpu.sync_copy(outbuf, out_hbm.at[cid])


@functools.partial(
    pl.kernel,
    out_type=jax.ShapeDtypeStruct((NC, 16), jnp.float32),
    mesh=plsc.VectorSubcoreMesh(core_axis_name="c", subcore_axis_name="s"),
    compiler_params=pltpu.CompilerParams(needs_layout_passes=False),
    scratch_types=[
        pltpu.VMEM((D, CHUNK), jnp.float32),
        pltpu.VMEM((D, CHUNK), jnp.float32),
        pltpu.VMEM((CHUNK,), jnp.int32),
        pltpu.VMEM((CHUNK,), jnp.int32),
        pltpu.VMEM((B_PER_CORE * SUMS_B,), jnp.float32),
        pltpu.VMEM((B_PER_CORE * 256,), jnp.float32),
        pltpu.VMEM((B_PER_CORE * 256,), jnp.float32),
        pltpu.VMEM((B_PER_CORE * ROW,), jnp.float32),
        pltpu.VMEM((B_PER_CORE * 16,), jnp.float32),
        pltpu.VMEM_SHARED((NS, B_PER_CORE * ROW), jnp.float32),
        pltpu.VMEM_SHARED((NS, B_PER_CORE * 16), jnp.float32),
        pltpu.VMEM((NS, B_PER_CORE * ROW), jnp.float32),
        pltpu.VMEM((NS, B_PER_CORE * 16), jnp.float32),
        pltpu.VMEM((B_PER_CORE * ROW,), jnp.float32),
        pltpu.VMEM((B_PER_CORE * ROW,), jnp.float32),
        pltpu.VMEM((16,), jnp.float32),
        pltpu.VMEM((16,), jnp.float32),
        pltpu.SemaphoreType.DMA,
        pltpu.SemaphoreType.DMA,
    ],
)
def _disc_loss_sc(emb_hbm, seg_hbm, out_hbm, *scratch):
    _body(emb_hbm, seg_hbm, out_hbm, *scratch)


def kernel(embedding, seg_gt):
    emb = embedding.reshape(BS, D, P)
    seg = seg_gt.reshape(BS, P).astype(jnp.int32)
    out = _disc_loss_sc(emb, seg)
    return out[0, 0] + out[1, 0]
